# baseline (speedup 1.0000x reference)
import functools

import jax
import jax.numpy as jnp
from jax import lax
from jax.experimental import pallas as pl
from jax.experimental.pallas import tpu as pltpu
from jax.experimental.pallas import tpu_sc as plsc

_B, _A, _MB, _NC, _NV, _NE = 4096, 38, 64, 128, 10, 5
_R = _A * _NV + 2 * _MB * _A + _MB * _NE
_RPAD = _R + 4
_L = 16
_NP = _NC // 2
_CG = 4
_PP = _NP // _CG
_BG = 8
_BSC = 2048
_BTC = _B - _BSC
_NBR = _BSC // _BG
_NBG = _NBR // _L
_TB = 256
_VW = 2 * _A
_EW = 3 * _MB
_OFF_E = _A * _NV
_OFF_T = _A * _NV + 2 * _MB * _A


def _prep_body(wv_ref, we_ref, wt_ref, w_ref, card_ref, t2_ref, bias_ref):
    gv = jnp.transpose(wv_ref[:].reshape(_NV * _A, _NC))
    ge = jnp.transpose(we_ref[:].reshape(_A * 2 * _MB, _NC))
    gt = jnp.transpose(wt_ref[:].reshape(_NE * _MB, _NC))
    gp = jnp.concatenate(
        [gv, ge, gt, jnp.zeros((_NC, _RPAD - _R), jnp.float32)],
        axis=1)
    lo = lax.bitcast_convert_type(
        gp[:_NP, :].astype(jnp.bfloat16), jnp.int16).astype(jnp.int32)
    hi = lax.bitcast_convert_type(
        gp[_NP:, :].astype(jnp.bfloat16), jnp.int16).astype(jnp.int32)
    t2_ref[:] = jnp.left_shift(hi, 16) | (lo & 0xFFFF)

    k_v = jax.nn.logsumexp(wv_ref[:], axis=0).sum(0)
    k_e = jax.nn.logsumexp(we_ref[:], axis=0).sum(0)
    k_t = jax.nn.logsumexp(wt_ref[:], axis=0).sum(0)
    w = w_ref[0, :]
    card = card_ref[:]
    card_s = card[_A - 1, _MB - 1] - jax.nn.logsumexp(
        jax.nn.logsumexp(card, axis=1))
    bias = w - jax.nn.logsumexp(w) - (k_v + k_e + k_t) + card_s
    bias_ref[0, :] = bias


_prep = pl.pallas_call(
    _prep_body,
    out_shape=[
        jax.ShapeDtypeStruct((_NP, _RPAD), jnp.int32),
        jax.ShapeDtypeStruct((1, _NC), jnp.float32),
    ],
)


def _finish_body(acc_ref, bias_ref, out_ref):
    x = acc_ref[:] + bias_ref[:]
    m = jnp.max(x, axis=0, keepdims=True)
    s = jnp.sum(jnp.exp(x - m), axis=0, keepdims=True)
    out_ref[:] = m + jnp.log(s)


_finish = pl.pallas_call(
    _finish_body,
    out_shape=jax.ShapeDtypeStruct((1, _BSC), jnp.float32),
)


def _dense_body(vt_ref, ed_ref, et_ref, wv_ref, we_ref, wt_ref, bias_ref,
                out_ref):
    acc = jnp.zeros((_TB, _NC), jnp.float32)
    vtb = vt_ref[:]
    for u in range(_NV):
        m = (vtb == u).astype(jnp.bfloat16)
        acc += jnp.dot(m, wv_ref[u], preferred_element_type=jnp.float32)
    edb = ed_ref[:]
    for u in range(_A):
        m = (edb == u).astype(jnp.bfloat16)
        acc += jnp.dot(m, we_ref[u], preferred_element_type=jnp.float32)
    etb = et_ref[:]
    for u in range(_NE):
        m = (etb == u).astype(jnp.bfloat16)
        acc += jnp.dot(m, wt_ref[u], preferred_element_type=jnp.float32)
    x = acc + bias_ref[:]
    mx = jnp.max(x, axis=1, keepdims=True)
    s = jnp.sum(jnp.exp(x - mx), axis=1, keepdims=True)
    out_ref[:] = mx + jnp.log(s)


_dense = pl.pallas_call(
    _dense_body,
    grid=(_BTC // _TB,),
    in_specs=[
        pl.BlockSpec((_TB, _A), lambda i: (i, 0)),
        pl.BlockSpec((_TB, 2 * _MB), lambda i: (i, 0)),
        pl.BlockSpec((_TB, _MB), lambda i: (i, 0)),
        pl.BlockSpec((_NV, _A, _NC), lambda i: (0, 0, 0)),
        pl.BlockSpec((_A, 2 * _MB, _NC), lambda i: (0, 0, 0)),
        pl.BlockSpec((_NE, _MB, _NC), lambda i: (0, 0, 0)),
        pl.BlockSpec((1, _NC), lambda i: (0, 0)),
    ],
    out_specs=pl.BlockSpec((_TB, 1), lambda i: (i, 0)),
    out_shape=jax.ShapeDtypeStruct((_BTC, 1), jnp.float32),
)


@functools.partial(
    pl.kernel,
    out_type=jax.ShapeDtypeStruct((_NC, _BSC), jnp.float32),
    mesh=plsc.VectorSubcoreMesh(core_axis_name="c", subcore_axis_name="s"),
    compiler_params=pltpu.CompilerParams(needs_layout_passes=False),
    name="sc_gather_sum",
    scratch_types=[
        pltpu.VMEM((_PP, _RPAD), jnp.int32),
        pltpu.VMEM((2, _L, _VW), jnp.int32),
        pltpu.VMEM((2, _L, _EW), jnp.int32),
        pltpu.VMEM((2 * _PP, _NBR), jnp.float32),
        pltpu.SemaphoreType.DMA,
        pltpu.SemaphoreType.DMA,
    ],
)
def _sc_main(v2_hbm, e2_hbm, t2_hbm, acct_hbm, tab_v, vv, ev, out_v,
             semv, seme):
    wid = lax.axis_index("s") * 2 + lax.axis_index("c")
    bg = wid // _CG
    cg = wid % _CG
    rbase = bg * _NBR
    pltpu.sync_copy(t2_hbm.at[pl.ds(cg * _PP, _PP)], tab_v)
    lanes = lax.iota(jnp.int32, _L)

    def gather_pairs(iv, accs):
        return tuple(
            accs[p] + plsc.bitcast(
                plsc.load_gather(tab_v, [jnp.full((_L,), p, jnp.int32), iv]),
                jnp.bfloat16)
            for p in range(_PP))

    def flush(t, accs, first):
        for p in range(_PP):
            flo, fhi = plsc.unpack(accs[p],
                                   format=plsc.PackFormat.INTERLEAVED)
            if first:
                out_v[p, pl.ds(t * _L, _L)] = flo
                out_v[p + _PP, pl.ds(t * _L, _L)] = fhi
            else:
                out_v[p, pl.ds(t * _L, _L)] = (
                    out_v[p, pl.ds(t * _L, _L)] + flo)
                out_v[p + _PP, pl.ds(t * _L, _L)] = (
                    out_v[p + _PP, pl.ds(t * _L, _L)] + fhi)

    def zero_accs():
        return tuple(jnp.zeros((2 * _L,), jnp.bfloat16)
                     for _ in range(_PP))

    def vcopy(t, par):
        return pltpu.make_async_copy(
            v2_hbm.at[pl.ds(rbase + t * _L, _L)], vv.at[par], semv)

    def ecopy(t, par):
        return pltpu.make_async_copy(
            e2_hbm.at[pl.ds(rbase + t * _L, _L)], ev.at[par], seme)

    def section(ref, par, n, col_fn, mul, base):
        def iv_at(j):
            val = plsc.load_gather(
                ref, [jnp.full((_L,), par, jnp.int32), lanes,
                      jnp.full((_L,), col_fn(j), jnp.int32)])
            return val * mul + (base + j)

        def body(j, carry):
            iv = carry[-1]
            iv_next = iv_at(jnp.minimum(j + 1, n - 1))
            return gather_pairs(iv, carry[:-1]) + (iv_next,)

        out = lax.fori_loop(0, n, body, zero_accs() + (iv_at(0),))
        return out[:-1]

    def b16_body(t, carry):
        par = t & 1
        vcopy(t, par).wait()
        ecopy(t, par).wait()
        tn = jnp.minimum(t + 1, _NBG - 1)
        vcopy(tn, 1 - par).start()
        ecopy(tn, 1 - par).start()

        flush(t, section(vv, par, _A, lambda j: 2 * j + 1, _A, 0), True)
        flush(t, section(ev, par, 2 * _MB,
                         lambda j: (j >> 1) * 3 + (j & 1),
                         2 * _MB, _OFF_E), False)
        flush(t, section(ev, par, _MB, lambda j: 3 * j + 2,
                         _MB, _OFF_T), False)
        return carry

    vcopy(0, 0).start()
    ecopy(0, 0).start()
    lax.fori_loop(0, _NBG, b16_body, 0)
    vcopy(_NBG - 1, _NBG & 1).wait()
    ecopy(_NBG - 1, _NBG & 1).wait()
    pltpu.sync_copy(out_v.at[pl.ds(0, _PP)],
                    acct_hbm.at[pl.ds(cg * _PP, _PP), pl.ds(rbase, _NBR)])
    pltpu.sync_copy(out_v.at[pl.ds(_PP, _PP)],
                    acct_hbm.at[pl.ds(_NP + cg * _PP, _PP),
                                pl.ds(rbase, _NBR)])


def kernel(v, e, logits_w, vtype_logits, edges_logits, etype_logits,
           card_logits):
    v = v.astype(jnp.int32)
    e = e.astype(jnp.int32)
    v2 = v.reshape(_B, _VW)
    e2 = e.reshape(_B, _EW)

    wv = jnp.transpose(vtype_logits, (2, 1, 0))
    we = jnp.transpose(edges_logits, (2, 1, 0))
    wt = jnp.transpose(etype_logits, (2, 1, 0))

    t2, bias2d = _prep(wv, we, wt, logits_w.reshape(1, _NC), card_logits)

    vt_tc = v[_BSC:, :, 1]
    ed_tc = e[_BSC:, :, :2].reshape(_BTC, 2 * _MB)
    et_tc = e[_BSC:, :, 2]

    acct = _sc_main(v2, e2, t2)
    out_tc = _dense(vt_tc, ed_tc, et_tc, wv.astype(jnp.bfloat16),
                    we.astype(jnp.bfloat16), wt.astype(jnp.bfloat16),
                    bias2d)
    out_sc = _finish(acct, bias2d.reshape(_NC, 1))
    return jnp.concatenate([out_sc.reshape(_BSC), out_tc.reshape(_BTC)])


# speedup vs baseline: 4.1266x; 4.1266x over previous
import functools

import jax
import jax.numpy as jnp
from jax import lax
from jax.experimental import pallas as pl
from jax.experimental.pallas import tpu as pltpu
from jax.experimental.pallas import tpu_sc as plsc

_B, _A, _MB, _NC, _NV, _NE = 4096, 38, 64, 128, 10, 5
_R = _A * _NV + 2 * _MB * _A + _MB * _NE
_RPAD = _R + 4
_L = 16
_NP = _NC // 2
_CG = 4
_PP = _NP // _CG
_BG = 8
_BSC = 2048
_BTC = _B - _BSC
_NBR = _BSC // _BG
_NBG = _NBR // _L
_TB = 256
_VW = 2 * _A
_EW = 3 * _MB
_OFF_E = _A * _NV
_OFF_T = _A * _NV + 2 * _MB * _A


def _prep_body(wv_ref, we_ref, wt_ref, w_ref, card_ref, t2_ref, bias_ref):
    gv = jnp.transpose(jnp.transpose(wv_ref[:], (1, 0, 2))
                       .reshape(_NV * _A, _NC))
    ge = jnp.transpose(jnp.transpose(we_ref[:], (1, 0, 2))
                       .reshape(_A * 2 * _MB, _NC))
    gt = jnp.transpose(jnp.transpose(wt_ref[:], (1, 0, 2))
                       .reshape(_NE * _MB, _NC))
    gp = jnp.concatenate(
        [gv, ge, gt, jnp.zeros((_NC, _RPAD - _R), jnp.float32)],
        axis=1)
    lo = lax.bitcast_convert_type(
        gp[:_NP, :].astype(jnp.bfloat16), jnp.int16).astype(jnp.int32)
    hi = lax.bitcast_convert_type(
        gp[_NP:, :].astype(jnp.bfloat16), jnp.int16).astype(jnp.int32)
    t2_ref[:] = jnp.left_shift(hi, 16) | (lo & 0xFFFF)

    k_v = jax.nn.logsumexp(wv_ref[:], axis=0).sum(0)
    k_e = jax.nn.logsumexp(we_ref[:], axis=0).sum(0)
    k_t = jax.nn.logsumexp(wt_ref[:], axis=0).sum(0)
    w = w_ref[0, :]
    card = card_ref[:]
    card_s = card[_A - 1, _MB - 1] - jax.nn.logsumexp(
        jax.nn.logsumexp(card, axis=1))
    bias = w - jax.nn.logsumexp(w) - (k_v + k_e + k_t) + card_s
    bias_ref[0, :] = bias


_prep = pl.pallas_call(
    _prep_body,
    out_shape=[
        jax.ShapeDtypeStruct((_NP, _RPAD), jnp.int32),
        jax.ShapeDtypeStruct((1, _NC), jnp.float32),
    ],
)


def _finish_body(acc_ref, bias_ref, out_ref):
    x = acc_ref[:] + bias_ref[:]
    m = jnp.max(x, axis=0, keepdims=True)
    s = jnp.sum(jnp.exp(x - m), axis=0, keepdims=True)
    out_ref[:] = m + jnp.log(s)


_finish = pl.pallas_call(
    _finish_body,
    out_shape=jax.ShapeDtypeStruct((1, _BSC), jnp.float32),
)


def _dense_body(vt_ref, ed_ref, et_ref, wv_ref, we_ref, wt_ref, bias_ref,
                out_ref):
    acc = jnp.zeros((_TB, _NC), jnp.float32)
    vtb = vt_ref[:]
    for u in range(_NV):
        m = (vtb == u).astype(jnp.bfloat16)
        acc += jnp.dot(m, wv_ref[u], preferred_element_type=jnp.float32)
    edb = ed_ref[:]
    for u in range(_A):
        m = (edb == u).astype(jnp.bfloat16)
        acc += jnp.dot(m, we_ref[u], preferred_element_type=jnp.float32)
    etb = et_ref[:]
    for u in range(_NE):
        m = (etb == u).astype(jnp.bfloat16)
        acc += jnp.dot(m, wt_ref[u], preferred_element_type=jnp.float32)
    x = acc + bias_ref[:]
    mx = jnp.max(x, axis=1, keepdims=True)
    s = jnp.sum(jnp.exp(x - mx), axis=1, keepdims=True)
    out_ref[:] = mx + jnp.log(s)


_dense = pl.pallas_call(
    _dense_body,
    grid=(_BTC // _TB,),
    in_specs=[
        pl.BlockSpec((_TB, _A), lambda i: (i, 0)),
        pl.BlockSpec((_TB, 2 * _MB), lambda i: (i, 0)),
        pl.BlockSpec((_TB, _MB), lambda i: (i, 0)),
        pl.BlockSpec((_NV, _A, _NC), lambda i: (0, 0, 0)),
        pl.BlockSpec((_A, 2 * _MB, _NC), lambda i: (0, 0, 0)),
        pl.BlockSpec((_NE, _MB, _NC), lambda i: (0, 0, 0)),
        pl.BlockSpec((1, _NC), lambda i: (0, 0)),
    ],
    out_specs=pl.BlockSpec((_TB, 1), lambda i: (i, 0)),
    out_shape=jax.ShapeDtypeStruct((_BTC, 1), jnp.float32),
)


@functools.partial(
    pl.kernel,
    out_type=jax.ShapeDtypeStruct((_NC, _BSC), jnp.float32),
    mesh=plsc.VectorSubcoreMesh(core_axis_name="c", subcore_axis_name="s"),
    compiler_params=pltpu.CompilerParams(needs_layout_passes=False),
    name="sc_gather_sum",
    scratch_types=[
        pltpu.VMEM((_PP, _RPAD), jnp.int32),
        pltpu.VMEM((2, _L, _VW), jnp.int32),
        pltpu.VMEM((2, _L, _EW), jnp.int32),
        pltpu.VMEM((2 * _PP, _NBR), jnp.float32),
        pltpu.SemaphoreType.DMA,
        pltpu.SemaphoreType.DMA,
    ],
)
def _sc_main(v2_hbm, e2_hbm, t2_hbm, acct_hbm, tab_v, vv, ev, out_v,
             semv, seme):
    wid = lax.axis_index("s") * 2 + lax.axis_index("c")
    bg = wid // _CG
    cg = wid % _CG
    rbase = bg * _NBR
    pltpu.sync_copy(t2_hbm.at[pl.ds(cg * _PP, _PP)], tab_v)
    lanes = lax.iota(jnp.int32, _L)

    def gather_pairs(iv, accs):
        return tuple(
            accs[p] + plsc.bitcast(
                plsc.load_gather(tab_v, [jnp.full((_L,), p, jnp.int32), iv]),
                jnp.bfloat16)
            for p in range(_PP))

    def flush(t, accs, first):
        for p in range(_PP):
            flo, fhi = plsc.unpack(accs[p],
                                   format=plsc.PackFormat.INTERLEAVED)
            if first:
                out_v[p, pl.ds(t * _L, _L)] = flo
                out_v[p + _PP, pl.ds(t * _L, _L)] = fhi
            else:
                out_v[p, pl.ds(t * _L, _L)] = (
                    out_v[p, pl.ds(t * _L, _L)] + flo)
                out_v[p + _PP, pl.ds(t * _L, _L)] = (
                    out_v[p + _PP, pl.ds(t * _L, _L)] + fhi)

    def zero_accs():
        return tuple(jnp.zeros((2 * _L,), jnp.bfloat16)
                     for _ in range(_PP))

    def vcopy(t, par):
        return pltpu.make_async_copy(
            v2_hbm.at[pl.ds(rbase + t * _L, _L)], vv.at[par], semv)

    def ecopy(t, par):
        return pltpu.make_async_copy(
            e2_hbm.at[pl.ds(rbase + t * _L, _L)], ev.at[par], seme)

    def section(ref, par, n, col_fn, off_fn):
        def iv_at(j):
            val = plsc.load_gather(
                ref, [jnp.full((_L,), par, jnp.int32), lanes,
                      jnp.full((_L,), col_fn(j), jnp.int32)])
            return val + off_fn(j)

        def body(j, carry):
            iv = carry[-1]
            iv_next = iv_at(jnp.minimum(j + 1, n - 1))
            return gather_pairs(iv, carry[:-1]) + (iv_next,)

        out = lax.fori_loop(0, n, body, zero_accs() + (iv_at(0),))
        return out[:-1]

    def b16_body(t, carry):
        par = t & 1
        vcopy(t, par).wait()
        ecopy(t, par).wait()
        tn = jnp.minimum(t + 1, _NBG - 1)
        vcopy(tn, 1 - par).start()
        ecopy(tn, 1 - par).start()

        flush(t, section(vv, par, _A, lambda j: 2 * j + 1,
                         lambda j: j * _NV), True)
        flush(t, section(ev, par, 2 * _MB,
                         lambda j: (j >> 1) * 3 + (j & 1),
                         lambda j: _OFF_E + j * _A), False)
        flush(t, section(ev, par, _MB, lambda j: 3 * j + 2,
                         lambda j: _OFF_T + j * _NE), False)
        return carry

    vcopy(0, 0).start()
    ecopy(0, 0).start()
    lax.fori_loop(0, _NBG, b16_body, 0)
    vcopy(_NBG - 1, _NBG & 1).wait()
    ecopy(_NBG - 1, _NBG & 1).wait()
    pltpu.sync_copy(out_v.at[pl.ds(0, _PP)],
                    acct_hbm.at[pl.ds(cg * _PP, _PP), pl.ds(rbase, _NBR)])
    pltpu.sync_copy(out_v.at[pl.ds(_PP, _PP)],
                    acct_hbm.at[pl.ds(_NP + cg * _PP, _PP),
                                pl.ds(rbase, _NBR)])


def kernel(v, e, logits_w, vtype_logits, edges_logits, etype_logits,
           card_logits):
    v = v.astype(jnp.int32)
    e = e.astype(jnp.int32)
    v2 = v.reshape(_B, _VW)
    e2 = e.reshape(_B, _EW)

    wv = jnp.transpose(vtype_logits, (2, 1, 0))
    we = jnp.transpose(edges_logits, (2, 1, 0))
    wt = jnp.transpose(etype_logits, (2, 1, 0))

    t2, bias2d = _prep(wv, we, wt, logits_w.reshape(1, _NC), card_logits)

    vt_tc = v[_BSC:, :, 1]
    ed_tc = e[_BSC:, :, :2].reshape(_BTC, 2 * _MB)
    et_tc = e[_BSC:, :, 2]

    acct = _sc_main(v2, e2, t2)
    out_tc = _dense(vt_tc, ed_tc, et_tc, wv.astype(jnp.bfloat16),
                    we.astype(jnp.bfloat16), wt.astype(jnp.bfloat16),
                    bias2d)
    out_sc = _finish(acct, bias2d.reshape(_NC, 1))
    return jnp.concatenate([out_sc.reshape(_BSC), out_tc.reshape(_BTC)])


# speedup vs baseline: 5.8451x; 1.4164x over previous
import functools

import jax
import jax.numpy as jnp
from jax import lax
from jax.experimental import pallas as pl
from jax.experimental.pallas import tpu as pltpu
from jax.experimental.pallas import tpu_sc as plsc

_B, _A, _MB, _NC, _NV, _NE = 4096, 38, 64, 128, 10, 5
_R = _A * _NV + 2 * _MB * _A + _MB * _NE
_RPAD = _R + 4
_L = 16
_NP = _NC // 2
_CG = 4
_PP = _NP // _CG
_BG = 8
_BSC = 1024
_BTC = _B - _BSC
_NBR = _BSC // _BG
_NBG = _NBR // _L
_TB = 256
_VW = 2 * _A
_EW = 3 * _MB
_OFF_E = _A * _NV
_OFF_T = _A * _NV + 2 * _MB * _A


def _prep_body(wv_ref, we_ref, wt_ref, w_ref, card_ref, t2_ref, bias_ref):
    gv = jnp.transpose(jnp.transpose(wv_ref[:], (1, 0, 2))
                       .reshape(_NV * _A, _NC))
    ge = jnp.transpose(jnp.transpose(we_ref[:], (1, 0, 2))
                       .reshape(_A * 2 * _MB, _NC))
    gt = jnp.transpose(jnp.transpose(wt_ref[:], (1, 0, 2))
                       .reshape(_NE * _MB, _NC))
    gp = jnp.concatenate(
        [gv, ge, gt, jnp.zeros((_NC, _RPAD - _R), jnp.float32)],
        axis=1)
    lo = lax.bitcast_convert_type(
        gp[:_NP, :].astype(jnp.bfloat16), jnp.int16).astype(jnp.int32)
    hi = lax.bitcast_convert_type(
        gp[_NP:, :].astype(jnp.bfloat16), jnp.int16).astype(jnp.int32)
    t2_ref[:] = jnp.left_shift(hi, 16) | (lo & 0xFFFF)

    k_v = jax.nn.logsumexp(wv_ref[:], axis=0).sum(0)
    k_e = jax.nn.logsumexp(we_ref[:], axis=0).sum(0)
    k_t = jax.nn.logsumexp(wt_ref[:], axis=0).sum(0)
    w = w_ref[0, :]
    card = card_ref[:]
    card_s = card[_A - 1, _MB - 1] - jax.nn.logsumexp(
        jax.nn.logsumexp(card, axis=1))
    bias = w - jax.nn.logsumexp(w) - (k_v + k_e + k_t) + card_s
    bias_ref[0, :] = bias


_prep = pl.pallas_call(
    _prep_body,
    out_shape=[
        jax.ShapeDtypeStruct((_NP, _RPAD), jnp.int32),
        jax.ShapeDtypeStruct((1, _NC), jnp.float32),
    ],
)


def _finish_body(acc_ref, bias_ref, out_ref):
    x = acc_ref[:] + bias_ref[:]
    m = jnp.max(x, axis=0, keepdims=True)
    s = jnp.sum(jnp.exp(x - m), axis=0, keepdims=True)
    out_ref[:] = m + jnp.log(s)


_finish = pl.pallas_call(
    _finish_body,
    out_shape=jax.ShapeDtypeStruct((1, _BSC), jnp.float32),
)


def _dense_body(vt_ref, ed_ref, et_ref, wv_ref, we_ref, wt_ref, bias_ref,
                out_ref):
    acc = jnp.zeros((_TB, _NC), jnp.float32)
    vtb = vt_ref[:]
    for u in range(_NV):
        m = (vtb == u).astype(jnp.bfloat16)
        acc += jnp.dot(m, wv_ref[u], preferred_element_type=jnp.float32)
    edb = ed_ref[:]
    for u in range(_A):
        m = (edb == u).astype(jnp.bfloat16)
        acc += jnp.dot(m, we_ref[u], preferred_element_type=jnp.float32)
    etb = et_ref[:]
    for u in range(_NE):
        m = (etb == u).astype(jnp.bfloat16)
        acc += jnp.dot(m, wt_ref[u], preferred_element_type=jnp.float32)
    x = acc + bias_ref[:]
    mx = jnp.max(x, axis=1, keepdims=True)
    s = jnp.sum(jnp.exp(x - mx), axis=1, keepdims=True)
    out_ref[:] = mx + jnp.log(s)


_dense = pl.pallas_call(
    _dense_body,
    grid=(_BTC // _TB,),
    in_specs=[
        pl.BlockSpec((_TB, _A), lambda i: (i, 0)),
        pl.BlockSpec((_TB, 2 * _MB), lambda i: (i, 0)),
        pl.BlockSpec((_TB, _MB), lambda i: (i, 0)),
        pl.BlockSpec((_NV, _A, _NC), lambda i: (0, 0, 0)),
        pl.BlockSpec((_A, 2 * _MB, _NC), lambda i: (0, 0, 0)),
        pl.BlockSpec((_NE, _MB, _NC), lambda i: (0, 0, 0)),
        pl.BlockSpec((1, _NC), lambda i: (0, 0)),
    ],
    out_specs=pl.BlockSpec((_TB, 1), lambda i: (i, 0)),
    out_shape=jax.ShapeDtypeStruct((_BTC, 1), jnp.float32),
)


@functools.partial(
    pl.kernel,
    out_type=jax.ShapeDtypeStruct((_NC, _BSC), jnp.float32),
    mesh=plsc.VectorSubcoreMesh(core_axis_name="c", subcore_axis_name="s"),
    compiler_params=pltpu.CompilerParams(needs_layout_passes=False),
    name="sc_gather_sum",
    scratch_types=[
        pltpu.VMEM((_PP, _RPAD), jnp.int32),
        pltpu.VMEM((2, _L, _VW), jnp.int32),
        pltpu.VMEM((2, _L, _EW), jnp.int32),
        pltpu.VMEM((2 * _PP, _NBR), jnp.float32),
        pltpu.SemaphoreType.DMA,
        pltpu.SemaphoreType.DMA,
    ],
)
def _sc_main(v2_hbm, e2_hbm, t2_hbm, acct_hbm, tab_v, vv, ev, out_v,
             semv, seme):
    wid = lax.axis_index("s") * 2 + lax.axis_index("c")
    bg = wid // _CG
    cg = wid % _CG
    rbase = bg * _NBR
    pltpu.sync_copy(t2_hbm.at[pl.ds(cg * _PP, _PP)], tab_v)
    lanes = lax.iota(jnp.int32, _L)

    def gather_pairs(iv, accs):
        return tuple(
            accs[p] + plsc.bitcast(
                plsc.load_gather(tab_v, [jnp.full((_L,), p, jnp.int32), iv]),
                jnp.bfloat16)
            for p in range(_PP))

    def flush(t, accs, first):
        for p in range(_PP):
            flo, fhi = plsc.unpack(accs[p],
                                   format=plsc.PackFormat.INTERLEAVED)
            if first:
                out_v[p, pl.ds(t * _L, _L)] = flo
                out_v[p + _PP, pl.ds(t * _L, _L)] = fhi
            else:
                out_v[p, pl.ds(t * _L, _L)] = (
                    out_v[p, pl.ds(t * _L, _L)] + flo)
                out_v[p + _PP, pl.ds(t * _L, _L)] = (
                    out_v[p + _PP, pl.ds(t * _L, _L)] + fhi)

    def zero_accs():
        return tuple(jnp.zeros((2 * _L,), jnp.bfloat16)
                     for _ in range(_PP))

    def vcopy(t, par):
        return pltpu.make_async_copy(
            v2_hbm.at[pl.ds(rbase + t * _L, _L)], vv.at[par], semv)

    def ecopy(t, par):
        return pltpu.make_async_copy(
            e2_hbm.at[pl.ds(rbase + t * _L, _L)], ev.at[par], seme)

    def section(ref, par, n, col_fn, off_fn):
        def iv_at(j):
            val = plsc.load_gather(
                ref, [jnp.full((_L,), par, jnp.int32), lanes,
                      jnp.full((_L,), col_fn(j), jnp.int32)])
            return val + off_fn(j)

        def body(j, carry):
            iv = carry[-1]
            iv_next = iv_at(jnp.minimum(j + 1, n - 1))
            return gather_pairs(iv, carry[:-1]) + (iv_next,)

        out = lax.fori_loop(0, n, body, zero_accs() + (iv_at(0),))
        return out[:-1]

    def b16_body(t, carry):
        par = t & 1
        vcopy(t, par).wait()
        ecopy(t, par).wait()
        tn = jnp.minimum(t + 1, _NBG - 1)
        vcopy(tn, 1 - par).start()
        ecopy(tn, 1 - par).start()

        flush(t, section(vv, par, _A, lambda j: 2 * j + 1,
                         lambda j: j * _NV), True)
        flush(t, section(ev, par, 2 * _MB,
                         lambda j: (j >> 1) * 3 + (j & 1),
                         lambda j: _OFF_E + j * _A), False)
        flush(t, section(ev, par, _MB, lambda j: 3 * j + 2,
                         lambda j: _OFF_T + j * _NE), False)
        return carry

    vcopy(0, 0).start()
    ecopy(0, 0).start()
    lax.fori_loop(0, _NBG, b16_body, 0)
    vcopy(_NBG - 1, _NBG & 1).wait()
    ecopy(_NBG - 1, _NBG & 1).wait()
    pltpu.sync_copy(out_v.at[pl.ds(0, _PP)],
                    acct_hbm.at[pl.ds(cg * _PP, _PP), pl.ds(rbase, _NBR)])
    pltpu.sync_copy(out_v.at[pl.ds(_PP, _PP)],
                    acct_hbm.at[pl.ds(_NP + cg * _PP, _PP),
                                pl.ds(rbase, _NBR)])


def kernel(v, e, logits_w, vtype_logits, edges_logits, etype_logits,
           card_logits):
    v = v.astype(jnp.int32)
    e = e.astype(jnp.int32)
    v2 = v.reshape(_B, _VW)
    e2 = e.reshape(_B, _EW)

    wv = jnp.transpose(vtype_logits, (2, 1, 0))
    we = jnp.transpose(edges_logits, (2, 1, 0))
    wt = jnp.transpose(etype_logits, (2, 1, 0))

    t2, bias2d = _prep(wv, we, wt, logits_w.reshape(1, _NC), card_logits)

    vt_tc = v[_BSC:, :, 1]
    ed_tc = e[_BSC:, :, :2].reshape(_BTC, 2 * _MB)
    et_tc = e[_BSC:, :, 2]

    acct = _sc_main(v2, e2, t2)
    out_tc = _dense(vt_tc, ed_tc, et_tc, wv.astype(jnp.bfloat16),
                    we.astype(jnp.bfloat16), wt.astype(jnp.bfloat16),
                    bias2d)
    out_sc = _finish(acct, bias2d.reshape(_NC, 1))
    return jnp.concatenate([out_sc.reshape(_BSC), out_tc.reshape(_BTC)])


# speedup vs baseline: 5.8499x; 1.0008x over previous
import functools

import jax
import jax.numpy as jnp
from jax import lax
from jax.experimental import pallas as pl
from jax.experimental.pallas import tpu as pltpu
from jax.experimental.pallas import tpu_sc as plsc

_B, _A, _MB, _NC, _NV, _NE = 4096, 38, 64, 128, 10, 5
_R = _A * _NV + 2 * _MB * _A + _MB * _NE
_RPAD = _R + 4
_L = 16
_NP = _NC // 2
_CG = 4
_PP = _NP // _CG
_BG = 8
_BSC = 1024
_BTC = _B - _BSC
_NBR = _BSC // _BG
_NBG = _NBR // _L
_TB = 512
_VW = 2 * _A
_EW = 3 * _MB
_OFF_E = _A * _NV
_OFF_T = _A * _NV + 2 * _MB * _A


def _prep_body(wv_ref, we_ref, wt_ref, w_ref, card_ref, t2_ref, bias_ref):
    gv = jnp.transpose(jnp.transpose(wv_ref[:], (1, 0, 2))
                       .reshape(_NV * _A, _NC))
    ge = jnp.transpose(jnp.transpose(we_ref[:], (1, 0, 2))
                       .reshape(_A * 2 * _MB, _NC))
    gt = jnp.transpose(jnp.transpose(wt_ref[:], (1, 0, 2))
                       .reshape(_NE * _MB, _NC))
    gp = jnp.concatenate(
        [gv, ge, gt, jnp.zeros((_NC, _RPAD - _R), jnp.float32)],
        axis=1)
    lo = lax.bitcast_convert_type(
        gp[:_NP, :].astype(jnp.bfloat16), jnp.int16).astype(jnp.int32)
    hi = lax.bitcast_convert_type(
        gp[_NP:, :].astype(jnp.bfloat16), jnp.int16).astype(jnp.int32)
    t2_ref[:] = jnp.left_shift(hi, 16) | (lo & 0xFFFF)

    k_v = jax.nn.logsumexp(wv_ref[:], axis=0).sum(0)
    k_e = jax.nn.logsumexp(we_ref[:], axis=0).sum(0)
    k_t = jax.nn.logsumexp(wt_ref[:], axis=0).sum(0)
    w = w_ref[0, :]
    card = card_ref[:]
    card_s = card[_A - 1, _MB - 1] - jax.nn.logsumexp(
        jax.nn.logsumexp(card, axis=1))
    bias = w - jax.nn.logsumexp(w) - (k_v + k_e + k_t) + card_s
    bias_ref[0, :] = bias


_prep = pl.pallas_call(
    _prep_body,
    out_shape=[
        jax.ShapeDtypeStruct((_NP, _RPAD), jnp.int32),
        jax.ShapeDtypeStruct((1, _NC), jnp.float32),
    ],
)


def _finish_body(acc_ref, bias_ref, out_ref):
    x = acc_ref[:] + bias_ref[:]
    m = jnp.max(x, axis=0, keepdims=True)
    s = jnp.sum(jnp.exp(x - m), axis=0, keepdims=True)
    out_ref[:] = m + jnp.log(s)


_finish = pl.pallas_call(
    _finish_body,
    out_shape=jax.ShapeDtypeStruct((1, _BSC), jnp.float32),
)


def _dense_body(vt_ref, ed_ref, et_ref, wv_ref, we_ref, wt_ref, bias_ref,
                out_ref):
    acc = jnp.zeros((_TB, _NC), jnp.float32)
    vtb = vt_ref[:]
    for u in range(_NV):
        m = (vtb == u).astype(jnp.bfloat16)
        acc += jnp.dot(m, wv_ref[u], preferred_element_type=jnp.float32)
    edb = ed_ref[:]
    for u in range(_A):
        m = (edb == u).astype(jnp.bfloat16)
        acc += jnp.dot(m, we_ref[u], preferred_element_type=jnp.float32)
    etb = et_ref[:]
    for u in range(_NE):
        m = (etb == u).astype(jnp.bfloat16)
        acc += jnp.dot(m, wt_ref[u], preferred_element_type=jnp.float32)
    x = acc + bias_ref[:]
    mx = jnp.max(x, axis=1, keepdims=True)
    s = jnp.sum(jnp.exp(x - mx), axis=1, keepdims=True)
    out_ref[:] = mx + jnp.log(s)


_dense = pl.pallas_call(
    _dense_body,
    grid=(_BTC // _TB,),
    in_specs=[
        pl.BlockSpec((_TB, _A), lambda i: (i, 0)),
        pl.BlockSpec((_TB, 2 * _MB), lambda i: (i, 0)),
        pl.BlockSpec((_TB, _MB), lambda i: (i, 0)),
        pl.BlockSpec((_NV, _A, _NC), lambda i: (0, 0, 0)),
        pl.BlockSpec((_A, 2 * _MB, _NC), lambda i: (0, 0, 0)),
        pl.BlockSpec((_NE, _MB, _NC), lambda i: (0, 0, 0)),
        pl.BlockSpec((1, _NC), lambda i: (0, 0)),
    ],
    out_specs=pl.BlockSpec((_TB, 1), lambda i: (i, 0)),
    out_shape=jax.ShapeDtypeStruct((_BTC, 1), jnp.float32),
)


@functools.partial(
    pl.kernel,
    out_type=jax.ShapeDtypeStruct((_NC, _BSC), jnp.float32),
    mesh=plsc.VectorSubcoreMesh(core_axis_name="c", subcore_axis_name="s"),
    compiler_params=pltpu.CompilerParams(needs_layout_passes=False),
    name="sc_gather_sum",
    scratch_types=[
        pltpu.VMEM((_PP, _RPAD), jnp.int32),
        pltpu.VMEM((2, _L, _VW), jnp.int32),
        pltpu.VMEM((2, _L, _EW), jnp.int32),
        pltpu.VMEM((2 * _PP, _NBR), jnp.float32),
        pltpu.SemaphoreType.DMA,
        pltpu.SemaphoreType.DMA,
    ],
)
def _sc_main(v2_hbm, e2_hbm, t2_hbm, acct_hbm, tab_v, vv, ev, out_v,
             semv, seme):
    wid = lax.axis_index("s") * 2 + lax.axis_index("c")
    bg = wid // _CG
    cg = wid % _CG
    rbase = bg * _NBR
    pltpu.sync_copy(t2_hbm.at[pl.ds(cg * _PP, _PP)], tab_v)
    lanes = lax.iota(jnp.int32, _L)

    def gather_pairs(iv, accs):
        return tuple(
            accs[p] + plsc.bitcast(
                plsc.load_gather(tab_v, [jnp.full((_L,), p, jnp.int32), iv]),
                jnp.bfloat16)
            for p in range(_PP))

    def flush(t, accs, first):
        for p in range(_PP):
            flo, fhi = plsc.unpack(accs[p],
                                   format=plsc.PackFormat.INTERLEAVED)
            if first:
                out_v[p, pl.ds(t * _L, _L)] = flo
                out_v[p + _PP, pl.ds(t * _L, _L)] = fhi
            else:
                out_v[p, pl.ds(t * _L, _L)] = (
                    out_v[p, pl.ds(t * _L, _L)] + flo)
                out_v[p + _PP, pl.ds(t * _L, _L)] = (
                    out_v[p + _PP, pl.ds(t * _L, _L)] + fhi)

    def zero_accs():
        return tuple(jnp.zeros((2 * _L,), jnp.bfloat16)
                     for _ in range(_PP))

    def vcopy(t, par):
        return pltpu.make_async_copy(
            v2_hbm.at[pl.ds(rbase + t * _L, _L)], vv.at[par], semv)

    def ecopy(t, par):
        return pltpu.make_async_copy(
            e2_hbm.at[pl.ds(rbase + t * _L, _L)], ev.at[par], seme)

    def section(ref, par, n, col_fn, off_fn):
        def iv_at(j):
            val = plsc.load_gather(
                ref, [jnp.full((_L,), par, jnp.int32), lanes,
                      jnp.full((_L,), col_fn(j), jnp.int32)])
            return val + off_fn(j)

        def body(j, carry):
            iv = carry[-1]
            iv_next = iv_at(jnp.minimum(j + 1, n - 1))
            return gather_pairs(iv, carry[:-1]) + (iv_next,)

        out = lax.fori_loop(0, n, body, zero_accs() + (iv_at(0),))
        return out[:-1]

    def b16_body(t, carry):
        par = t & 1
        vcopy(t, par).wait()
        ecopy(t, par).wait()
        tn = jnp.minimum(t + 1, _NBG - 1)
        vcopy(tn, 1 - par).start()
        ecopy(tn, 1 - par).start()

        flush(t, section(vv, par, _A, lambda j: 2 * j + 1,
                         lambda j: j * _NV), True)
        flush(t, section(ev, par, 2 * _MB,
                         lambda j: (j >> 1) * 3 + (j & 1),
                         lambda j: _OFF_E + j * _A), False)
        flush(t, section(ev, par, _MB, lambda j: 3 * j + 2,
                         lambda j: _OFF_T + j * _NE), False)
        return carry

    vcopy(0, 0).start()
    ecopy(0, 0).start()
    lax.fori_loop(0, _NBG, b16_body, 0)
    vcopy(_NBG - 1, _NBG & 1).wait()
    ecopy(_NBG - 1, _NBG & 1).wait()
    pltpu.sync_copy(out_v.at[pl.ds(0, _PP)],
                    acct_hbm.at[pl.ds(cg * _PP, _PP), pl.ds(rbase, _NBR)])
    pltpu.sync_copy(out_v.at[pl.ds(_PP, _PP)],
                    acct_hbm.at[pl.ds(_NP + cg * _PP, _PP),
                                pl.ds(rbase, _NBR)])


def kernel(v, e, logits_w, vtype_logits, edges_logits, etype_logits,
           card_logits):
    v = v.astype(jnp.int32)
    e = e.astype(jnp.int32)
    v2 = v.reshape(_B, _VW)
    e2 = e.reshape(_B, _EW)

    wv = jnp.transpose(vtype_logits, (2, 1, 0))
    we = jnp.transpose(edges_logits, (2, 1, 0))
    wt = jnp.transpose(etype_logits, (2, 1, 0))

    t2, bias2d = _prep(wv, we, wt, logits_w.reshape(1, _NC), card_logits)

    vt_tc = v[_BSC:, :, 1]
    ed_tc = e[_BSC:, :, :2].reshape(_BTC, 2 * _MB)
    et_tc = e[_BSC:, :, 2]

    acct = _sc_main(v2, e2, t2)
    out_tc = _dense(vt_tc, ed_tc, et_tc, wv.astype(jnp.bfloat16),
                    we.astype(jnp.bfloat16), wt.astype(jnp.bfloat16),
                    bias2d)
    out_sc = _finish(acct, bias2d.reshape(_NC, 1))
    return jnp.concatenate([out_sc.reshape(_BSC), out_tc.reshape(_BTC)])


# speedup vs baseline: 6.7434x; 1.1528x over previous
import functools

import jax
import jax.numpy as jnp
from jax import lax
from jax.experimental import pallas as pl
from jax.experimental.pallas import tpu as pltpu
from jax.experimental.pallas import tpu_sc as plsc

_B, _A, _MB, _NC, _NV, _NE = 4096, 38, 64, 128, 10, 5
_R = _A * _NV + 2 * _MB * _A + _MB * _NE
_RPAD = _R + 4
_L = 16
_NP = _NC // 2
_CG = 4
_PP = _NP // _CG
_BG = 8
_BSC = 1024
_BTC = _B - _BSC
_NBR = _BSC // _BG
_NBG = _NBR // _L
_TB = 512
_VW = 2 * _A
_EW = 3 * _MB
_OFF_E = _A * _NV
_OFF_T = _A * _NV + 2 * _MB * _A


def _prep_body(wv_ref, we_ref, wt_ref, w_ref, card_ref, t2_ref, bias_ref):
    gv = jnp.transpose(jnp.transpose(wv_ref[:], (1, 0, 2))
                       .reshape(_NV * _A, _NC))
    ge = jnp.transpose(jnp.transpose(we_ref[:], (1, 0, 2))
                       .reshape(_A * 2 * _MB, _NC))
    gt = jnp.transpose(jnp.transpose(wt_ref[:], (1, 0, 2))
                       .reshape(_NE * _MB, _NC))
    gp = jnp.concatenate(
        [gv, ge, gt, jnp.zeros((_NC, _RPAD - _R), jnp.float32)],
        axis=1)
    lo = lax.bitcast_convert_type(
        gp[:_NP, :].astype(jnp.bfloat16), jnp.int16).astype(jnp.int32)
    hi = lax.bitcast_convert_type(
        gp[_NP:, :].astype(jnp.bfloat16), jnp.int16).astype(jnp.int32)
    t2_ref[:] = jnp.left_shift(hi, 16) | (lo & 0xFFFF)

    k_v = jax.nn.logsumexp(wv_ref[:], axis=0).sum(0)
    k_e = jax.nn.logsumexp(we_ref[:], axis=0).sum(0)
    k_t = jax.nn.logsumexp(wt_ref[:], axis=0).sum(0)
    w = w_ref[0, :]
    card = card_ref[:]
    card_s = card[_A - 1, _MB - 1] - jax.nn.logsumexp(
        jax.nn.logsumexp(card, axis=1))
    bias = w - jax.nn.logsumexp(w) - (k_v + k_e + k_t) + card_s
    bias_ref[0, :] = bias


_prep = pl.pallas_call(
    _prep_body,
    out_shape=[
        jax.ShapeDtypeStruct((_NP, _RPAD), jnp.int32),
        jax.ShapeDtypeStruct((1, _NC), jnp.float32),
    ],
)


def _finish_body(acc_ref, bias_ref, out_ref):
    x = acc_ref[:] + bias_ref[:]
    m = jnp.max(x, axis=0, keepdims=True)
    s = jnp.sum(jnp.exp(x - m), axis=0, keepdims=True)
    out_ref[:] = m + jnp.log(s)


_finish = pl.pallas_call(
    _finish_body,
    out_shape=jax.ShapeDtypeStruct((1, _BSC), jnp.float32),
)


def _dense_body(vt_ref, ed_ref, et_ref, wv_ref, we_ref, wt_ref, bias_ref,
                out_ref):
    acc = jnp.zeros((_TB, _NC), jnp.float32)
    vtb = vt_ref[:]
    for u in range(_NV):
        m = (vtb == u).astype(jnp.bfloat16)
        acc += jnp.dot(m, wv_ref[u], preferred_element_type=jnp.float32)
    edb = ed_ref[:]
    for u in range(_A):
        m = (edb == u).astype(jnp.bfloat16)
        acc += jnp.dot(m, we_ref[u], preferred_element_type=jnp.float32)
    etb = et_ref[:]
    for u in range(_NE):
        m = (etb == u).astype(jnp.bfloat16)
        acc += jnp.dot(m, wt_ref[u], preferred_element_type=jnp.float32)
    x = acc + bias_ref[:]
    mx = jnp.max(x, axis=1, keepdims=True)
    s = jnp.sum(jnp.exp(x - mx), axis=1, keepdims=True)
    out_ref[:] = mx + jnp.log(s)


_dense = pl.pallas_call(
    _dense_body,
    grid=(_BTC // _TB,),
    in_specs=[
        pl.BlockSpec((_TB, _A), lambda i: (i, 0)),
        pl.BlockSpec((_TB, 2 * _MB), lambda i: (i, 0)),
        pl.BlockSpec((_TB, _MB), lambda i: (i, 0)),
        pl.BlockSpec((_NV, _A, _NC), lambda i: (0, 0, 0)),
        pl.BlockSpec((_A, 2 * _MB, _NC), lambda i: (0, 0, 0)),
        pl.BlockSpec((_NE, _MB, _NC), lambda i: (0, 0, 0)),
        pl.BlockSpec((1, _NC), lambda i: (0, 0)),
    ],
    out_specs=pl.BlockSpec((_TB, 1), lambda i: (i, 0)),
    out_shape=jax.ShapeDtypeStruct((_BTC, 1), jnp.float32),
)


@functools.partial(
    pl.kernel,
    out_type=jax.ShapeDtypeStruct((_NC, _BSC), jnp.float32),
    mesh=plsc.VectorSubcoreMesh(core_axis_name="c", subcore_axis_name="s"),
    compiler_params=pltpu.CompilerParams(needs_layout_passes=False),
    name="sc_gather_sum",
    scratch_types=[
        pltpu.VMEM((_PP, _RPAD), jnp.int32),
        pltpu.VMEM((2, _L, _VW), jnp.int32),
        pltpu.VMEM((2, _L, _EW), jnp.int32),
        pltpu.VMEM((2 * _PP, _NBR), jnp.float32),
        pltpu.SemaphoreType.DMA,
        pltpu.SemaphoreType.DMA,
    ],
)
def _sc_main(v2_hbm, e2_hbm, t2_hbm, acct_hbm, tab_v, vv, ev, out_v,
             semv, seme):
    wid = lax.axis_index("s") * 2 + lax.axis_index("c")
    bg = wid // _CG
    cg = wid % _CG
    rbase = bg * _NBR
    pltpu.sync_copy(t2_hbm.at[pl.ds(cg * _PP, _PP)], tab_v)
    lanes = lax.iota(jnp.int32, _L)

    def gather_pairs(iv, accs):
        return tuple(
            accs[p] + plsc.bitcast(
                plsc.load_gather(tab_v, [jnp.full((_L,), p, jnp.int32), iv]),
                jnp.bfloat16)
            for p in range(_PP))

    def flush(t, accs, first):
        for p in range(_PP):
            flo, fhi = plsc.unpack(accs[p],
                                   format=plsc.PackFormat.INTERLEAVED)
            if first:
                out_v[p, pl.ds(t * _L, _L)] = flo
                out_v[p + _PP, pl.ds(t * _L, _L)] = fhi
            else:
                out_v[p, pl.ds(t * _L, _L)] = (
                    out_v[p, pl.ds(t * _L, _L)] + flo)
                out_v[p + _PP, pl.ds(t * _L, _L)] = (
                    out_v[p + _PP, pl.ds(t * _L, _L)] + fhi)

    def zero_accs():
        return tuple(jnp.zeros((2 * _L,), jnp.bfloat16)
                     for _ in range(_PP))

    def vcopy(t, par):
        return pltpu.make_async_copy(
            v2_hbm.at[pl.ds(rbase + t * _L, _L)], vv.at[par], semv)

    def ecopy(t, par):
        return pltpu.make_async_copy(
            e2_hbm.at[pl.ds(rbase + t * _L, _L)], ev.at[par], seme)

    def section(ref, par, n, col_fn, off_fn):
        def iv_at(j):
            val = plsc.load_gather(
                ref, [jnp.full((_L,), par, jnp.int32), lanes,
                      jnp.full((_L,), col_fn(j), jnp.int32)])
            return val + off_fn(j)

        def body(j, carry):
            iv = carry[-1]
            iv_next = iv_at(jnp.minimum(j + 1, n - 1))
            return gather_pairs(iv, carry[:-1]) + (iv_next,)

        out = lax.fori_loop(0, n, body, zero_accs() + (iv_at(0),))
        return out[:-1]

    def b16_body(t, carry):
        par = t & 1
        vcopy(t, par).wait()
        ecopy(t, par).wait()
        tn = jnp.minimum(t + 1, _NBG - 1)
        vcopy(tn, 1 - par).start()
        ecopy(tn, 1 - par).start()

        flush(t, section(vv, par, _A, lambda j: 2 * j + 1,
                         lambda j: j * _NV), True)
        flush(t, section(ev, par, 2 * _MB,
                         lambda j: (j >> 1) * 3 + (j & 1),
                         lambda j: _OFF_E + j * _A), False)
        flush(t, section(ev, par, _MB, lambda j: 3 * j + 2,
                         lambda j: _OFF_T + j * _NE), False)
        return carry

    vcopy(0, 0).start()
    ecopy(0, 0).start()
    lax.fori_loop(0, _NBG, b16_body, 0)
    vcopy(_NBG - 1, _NBG & 1).wait()
    ecopy(_NBG - 1, _NBG & 1).wait()
    pltpu.sync_copy(out_v.at[pl.ds(0, _PP)],
                    acct_hbm.at[pl.ds(cg * _PP, _PP), pl.ds(rbase, _NBR)])
    pltpu.sync_copy(out_v.at[pl.ds(_PP, _PP)],
                    acct_hbm.at[pl.ds(_NP + cg * _PP, _PP),
                                pl.ds(rbase, _NBR)])


def kernel(v, e, logits_w, vtype_logits, edges_logits, etype_logits,
           card_logits):
    v = v.astype(jnp.int32)
    e = e.astype(jnp.int32)
    v2 = v[:_BSC].reshape(_BSC, _VW)
    e2 = e[:_BSC].reshape(_BSC, _EW)

    wv = jnp.transpose(vtype_logits, (2, 1, 0))
    we = jnp.transpose(edges_logits, (2, 1, 0))
    wt = jnp.transpose(etype_logits, (2, 1, 0))

    t2, bias2d = _prep(wv, we, wt, logits_w.reshape(1, _NC), card_logits)

    vt_tc = v[_BSC:, :, 1]
    ed_tc = e[_BSC:, :, :2].reshape(_BTC, 2 * _MB)
    et_tc = e[_BSC:, :, 2]

    acct = _sc_main(v2, e2, t2)
    out_tc = _dense(vt_tc, ed_tc, et_tc, wv.astype(jnp.bfloat16),
                    we.astype(jnp.bfloat16), wt.astype(jnp.bfloat16),
                    bias2d)
    out_sc = _finish(acct, bias2d.reshape(_NC, 1))
    return jnp.concatenate([out_sc.reshape(_BSC), out_tc.reshape(_BTC)])
